# baseline (speedup 1.0000x reference)
import functools

import jax
import jax.numpy as jnp
from jax import lax
from jax.experimental import pallas as pl
from jax.experimental.pallas import tpu as pltpu


def _pos_body(B, H, W, row_ref, col_ref, out_ref, scratch, sem):
    F = row_ref.shape[1]
    HW = H * W
    lane_w = lax.broadcasted_iota(jnp.int32, (W, HW), 1)
    sub_w = lax.broadcasted_iota(jnp.int32, (W, HW), 0)
    tile_sel = (lane_w % W == sub_w).astype(jnp.float32)
    lane_h = lax.broadcasted_iota(jnp.int32, (H, HW), 1)
    sub_h = lax.broadcasted_iota(jnp.int32, (H, HW), 0)
    rep_sel = (lane_h // W == sub_h).astype(jnp.float32)
    dn = (((0,), (0,)), ((), ()))
    scratch[0, :F] = lax.dot_general(
        col_ref[:W, :], tile_sel, dn, preferred_element_type=jnp.float32)
    scratch[0, F:] = lax.dot_general(
        row_ref[:H, :], rep_sel, dn, preferred_element_type=jnp.float32)
    plane = scratch[0]
    for b in range(1, B):
        scratch[b] = plane
    pltpu.make_async_copy(scratch, out_ref, sem).start()
    pltpu.make_async_copy(scratch, out_ref, sem).wait()


def kernel(input, row_embed, col_embed):
    B, C, H, W = input.shape
    N, F = row_embed.shape
    out = pl.pallas_call(
        functools.partial(_pos_body, B, H, W),
        in_specs=[
            pl.BlockSpec(memory_space=pltpu.MemorySpace.VMEM),
            pl.BlockSpec(memory_space=pltpu.MemorySpace.VMEM),
        ],
        out_specs=pl.BlockSpec(memory_space=pltpu.MemorySpace.HBM),
        out_shape=jax.ShapeDtypeStruct((B, 2 * F, H * W), row_embed.dtype),
        scratch_shapes=[
            pltpu.VMEM((B, 2 * F, H * W), jnp.float32),
            pltpu.SemaphoreType.DMA,
        ],
    )(row_embed, col_embed)
    return out.reshape(B, 2 * F, H, W)


# speedup vs baseline: 1.0762x; 1.0762x over previous
import functools

import jax
import jax.numpy as jnp
from jax import lax
from jax.experimental import pallas as pl
from jax.experimental.pallas import tpu as pltpu

_NSEM = 2


def _pos_body(B, H, W, row_ref, col_ref, out_ref, scratch, sems):
    F = row_ref.shape[1]
    HW = H * W
    lane_w = lax.broadcasted_iota(jnp.int32, (W, HW), 1)
    sub_w = lax.broadcasted_iota(jnp.int32, (W, HW), 0)
    tile_sel = (lane_w % W == sub_w).astype(jnp.float32)
    lane_h = lax.broadcasted_iota(jnp.int32, (H, HW), 1)
    sub_h = lax.broadcasted_iota(jnp.int32, (H, HW), 0)
    rep_sel = (lane_h // W == sub_h).astype(jnp.float32)
    dn = (((0,), (0,)), ((), ()))
    scratch[:F] = lax.dot_general(
        col_ref[:W, :], tile_sel, dn, preferred_element_type=jnp.float32)
    scratch[F:] = lax.dot_general(
        row_ref[:H, :], rep_sel, dn, preferred_element_type=jnp.float32)
    for b in range(B):
        pltpu.make_async_copy(
            scratch, out_ref.at[b], sems.at[b % _NSEM]).start(
                priority=b % _NSEM)
    for b in range(B):
        pltpu.make_async_copy(
            scratch, out_ref.at[b], sems.at[b % _NSEM]).wait()


def kernel(input, row_embed, col_embed):
    B, C, H, W = input.shape
    N, F = row_embed.shape
    out = pl.pallas_call(
        functools.partial(_pos_body, B, H, W),
        in_specs=[
            pl.BlockSpec(memory_space=pltpu.MemorySpace.VMEM),
            pl.BlockSpec(memory_space=pltpu.MemorySpace.VMEM),
        ],
        out_specs=pl.BlockSpec(memory_space=pltpu.MemorySpace.HBM),
        out_shape=jax.ShapeDtypeStruct((B, 2 * F, H * W), row_embed.dtype),
        scratch_shapes=[
            pltpu.VMEM((2 * F, H * W), jnp.float32),
            pltpu.SemaphoreType.DMA((_NSEM,)),
        ],
    )(row_embed, col_embed)
    return out.reshape(B, 2 * F, H, W)
